# baseline (speedup 1.0000x reference)
import functools

import jax
import jax.numpy as jnp
from jax.experimental import pallas as pl
from jax.experimental.pallas import tpu as pltpu

_D = 16
_H = 20
_PACK = 128 // _D
_HP = _PACK * _H
_RPO = 128 // _PACK


def _fused_mlp_kernel(x3_hbm, w1p_ref, b1p_ref, w2p_ref, b2p_ref, out_ref,
                      xs, in_sem, *, steps, tb):
    i = pl.program_id(0)

    def start_in(slot, blk):
        for s in range(_PACK):
            pltpu.make_async_copy(
                x3_hbm.at[pl.ds(blk * tb, tb), s, :],
                xs.at[slot, s], in_sem.at[slot],
            ).start()

    def wait_in(slot):
        pltpu.make_async_copy(xs.at[slot], xs.at[slot],
                              in_sem.at[slot]).wait()

    @pl.when(i == 0)
    def _():
        start_in(0, 0)
        if steps > 1:
            start_in(1, 1)
        if steps > 2:
            start_in(2, 2)

    @pl.when(i + 3 < steps)
    def _():
        start_in(jax.lax.rem(i + 3, 4), i + 3)

    cur = jax.lax.rem(i, 4)
    wait_in(cur)

    xp = jnp.concatenate([xs[cur, s] for s in range(_PACK)], axis=-1)
    xb = xp.astype(jnp.bfloat16)
    h = jnp.dot(xb, w1p_ref[...],
                preferred_element_type=jnp.float32)
    h = jnp.maximum(h + b1p_ref[...], 0.0)
    orep = jnp.dot(h.astype(jnp.bfloat16), w2p_ref[...],
                   preferred_element_type=jnp.float32)
    o3 = orep.reshape(tb // _RPO, _RPO, 128)
    m = jax.lax.broadcasted_iota(jnp.int32, (1, _RPO, 128), 1)
    c = jax.lax.broadcasted_iota(jnp.int32, (1, _RPO, 128), 2)
    sel = (c // _PACK) == m
    out = jnp.sum(jnp.where(sel, o3, 0.0), axis=1)
    out_ref[...] = out + b2p_ref[...]


def _pick_tb(rows):
    for tb in (2048, 1024, 512, 256, 128, 64, 32, 16):
        if rows % tb == 0:
            return tb
    return rows


def kernel(x, w1T, w2T):
    B, D = x.shape
    assert D == _D, (x.shape,)
    w1 = w1T[:_H, :_D].T
    b1 = w1T[:_H, _D]
    w2c = w2T[0, :_H]
    b2 = w2T[0, _H]

    eye = jnp.eye(_PACK, dtype=jnp.float32)
    w1p = jnp.kron(eye, w1).astype(jnp.bfloat16)
    b1p = jnp.tile(b1, _PACK).reshape(1, _HP)
    w2p = jnp.tile(jnp.kron(eye, w2c.reshape(_H, 1)),
                   (1, _RPO)).astype(jnp.bfloat16)
    b2p = jnp.full((1, 128), b2, jnp.float32)

    chunk = _PACK * _RPO * 8
    Bp = ((B + chunk - 1) // chunk) * chunk
    if Bp != B:
        x = jnp.pad(x, ((0, Bp - B), (0, 0)))
    rows = Bp // _PACK
    x3 = x.reshape(rows, _PACK, _D)

    tb = _pick_tb(rows)
    steps = rows // tb
    body = functools.partial(_fused_mlp_kernel, steps=steps, tb=tb)
    out = pl.pallas_call(
        body,
        out_shape=jax.ShapeDtypeStruct((rows // _RPO, 128), jnp.float32),
        grid=(steps,),
        in_specs=[
            pl.BlockSpec(memory_space=pltpu.MemorySpace.HBM),
            pl.BlockSpec((128, _HP), lambda i: (0, 0)),
            pl.BlockSpec((1, _HP), lambda i: (0, 0)),
            pl.BlockSpec((_HP, 128), lambda i: (0, 0)),
            pl.BlockSpec((1, 128), lambda i: (0, 0)),
        ],
        out_specs=pl.BlockSpec((tb // _RPO, 128), lambda i: (i, 0)),
        scratch_shapes=[
            pltpu.VMEM((4, _PACK, tb, _D), jnp.float32),
            pltpu.SemaphoreType.DMA((4,)),
        ],
        compiler_params=pltpu.CompilerParams(
            dimension_semantics=("arbitrary",),
        ),
    )(x3, w1p, b1p, w2p, b2p)

    return out.reshape(Bp, 1)[:B].astype(x.dtype)


# speedup vs baseline: 1.0080x; 1.0080x over previous
import functools

import jax
import jax.numpy as jnp
from jax.experimental import pallas as pl
from jax.experimental.pallas import tpu as pltpu

_D = 16
_H = 20
_PACK = 128 // _D
_HP = _PACK * _H
_RPO = 128 // _PACK


def _fused_mlp_kernel(x3_hbm, w1p_ref, b1p_ref, w2p_ref, b2p_ref, out_ref,
                      xs, in_sem, *, steps, tb):
    i = pl.program_id(0)

    def start_in(slot, blk):
        for s in range(_PACK):
            pltpu.make_async_copy(
                x3_hbm.at[pl.ds(blk * tb, tb), s, :],
                xs.at[slot, s], in_sem.at[slot],
            ).start()

    def wait_in(slot):
        pltpu.make_async_copy(xs.at[slot], xs.at[slot],
                              in_sem.at[slot]).wait()

    @pl.when(i == 0)
    def _():
        start_in(0, 0)
        if steps > 1:
            start_in(1, 1)

    @pl.when(i + 2 < steps)
    def _():
        start_in(jax.lax.rem(i + 2, 3), i + 2)

    cur = jax.lax.rem(i, 3)
    wait_in(cur)

    xp = jnp.concatenate([xs[cur, s] for s in range(_PACK)], axis=-1)
    xb = xp.astype(jnp.bfloat16)
    h = jnp.dot(xb, w1p_ref[...],
                preferred_element_type=jnp.float32)
    h = jnp.maximum(h + b1p_ref[...], 0.0)
    orep = jnp.dot(h.astype(jnp.bfloat16), w2p_ref[...],
                   preferred_element_type=jnp.float32)
    o3 = orep.reshape(tb // _RPO, _RPO, 128)
    m = jax.lax.broadcasted_iota(jnp.int32, (1, _RPO, 128), 1)
    c = jax.lax.broadcasted_iota(jnp.int32, (1, _RPO, 128), 2)
    sel = (c // _PACK) == m
    out = jnp.sum(jnp.where(sel, o3, 0.0), axis=1)
    out_ref[...] = out + b2p_ref[...]


def _pick_tb(rows):
    for tb in (2048, 1024, 512, 256, 128, 64, 32, 16):
        if rows % tb == 0:
            return tb
    return rows


def kernel(x, w1T, w2T):
    B, D = x.shape
    assert D == _D, (x.shape,)
    w1 = w1T[:_H, :_D].T
    b1 = w1T[:_H, _D]
    w2c = w2T[0, :_H]
    b2 = w2T[0, _H]

    eye = jnp.eye(_PACK, dtype=jnp.float32)
    w1p = jnp.kron(eye, w1).astype(jnp.bfloat16)
    b1p = jnp.tile(b1, _PACK).reshape(1, _HP)
    w2p = jnp.tile(jnp.kron(eye, w2c.reshape(_H, 1)),
                   (1, _RPO)).astype(jnp.bfloat16)
    b2p = jnp.full((1, 128), b2, jnp.float32)

    chunk = _PACK * _RPO * 8
    Bp = ((B + chunk - 1) // chunk) * chunk
    if Bp != B:
        x = jnp.pad(x, ((0, Bp - B), (0, 0)))
    rows = Bp // _PACK
    x3 = x.reshape(rows, _PACK, _D)

    tb = _pick_tb(rows)
    steps = rows // tb
    body = functools.partial(_fused_mlp_kernel, steps=steps, tb=tb)
    out = pl.pallas_call(
        body,
        out_shape=jax.ShapeDtypeStruct((rows // _RPO, 128), jnp.float32),
        grid=(steps,),
        in_specs=[
            pl.BlockSpec(memory_space=pltpu.MemorySpace.HBM),
            pl.BlockSpec((128, _HP), lambda i: (0, 0)),
            pl.BlockSpec((1, _HP), lambda i: (0, 0)),
            pl.BlockSpec((_HP, 128), lambda i: (0, 0)),
            pl.BlockSpec((1, 128), lambda i: (0, 0)),
        ],
        out_specs=pl.BlockSpec((tb // _RPO, 128), lambda i: (i, 0)),
        scratch_shapes=[
            pltpu.VMEM((3, _PACK, tb, _D), jnp.float32),
            pltpu.SemaphoreType.DMA((3,)),
        ],
        compiler_params=pltpu.CompilerParams(
            dimension_semantics=("arbitrary",),
        ),
    )(x3, w1p, b1p, w2p, b2p)

    return out.reshape(Bp, 1)[:B].astype(x.dtype)
